# baseline (speedup 1.0000x reference)
import functools

import jax
import jax.numpy as jnp
from jax import lax
from jax.experimental import pallas as pl
from jax.experimental.pallas import tpu as pltpu
from jax.experimental.pallas import tpu_sc as plsc

N = 10000
E = 320000
H = 128
NF6 = 6
OUT_D = 264

NC = 2
NS = 16
NW = NC * NS
EPW = E // NW
B = 80
NCHUNK = EPW // B
L = 16


def _rsqrt(x):
    xi = lax.bitcast_convert_type(x, jnp.int32)
    yi = jnp.int32(0x5F3759DF) - lax.shift_right_logical(xi, 1)
    y = lax.bitcast_convert_type(yi, jnp.float32)
    xh = x * 0.5
    for _ in range(3):
        y = y * (1.5 - xh * y * y)
    return y


def _edge_body(row_hbm, col_hbm, weight_hbm, emb_hbm, feat6_hbm, out_hbm,
               ir0, ir1, ic0, ic1, wv0, wv1, feat6, pack0, pack1,
               sem_emb0, sem_emb1, sem_out0, sem_out1, sem_i0, sem_i1):
    wid = lax.axis_index("s") * NC + lax.axis_index("c")
    base0 = wid * EPW
    pltpu.sync_copy(feat6_hbm, feat6)

    i16 = lax.iota(jnp.int32, L)

    def _normalize(grp, carry):
        n6 = (i16 + grp * L) * NF6
        c = [plsc.load_gather(feat6, [n6 + k]) for k in range(4)]
        ssum = c[0] * c[0] + c[1] * c[1] + c[2] * c[2] + c[3] * c[3]
        inv = _rsqrt(jnp.maximum(ssum, 1e-16))
        for k in range(4):
            plsc.store_scatter(feat6, [n6 + k], c[k] * inv)
        return carry

    lax.fori_loop(0, N // L, _normalize, 0)

    idx = [(ir0, ic0, wv0, sem_i0), (ir1, ic1, wv1, sem_i1)]
    packs = [(pack0, sem_emb0, sem_out0), (pack1, sem_emb1, sem_out1)]

    def prefetch(g, s):
        ir, ic, wv, sem = idx[s]
        b = jnp.minimum(base0 + g * B, base0 + EPW - B)
        pltpu.async_copy(row_hbm.at[pl.ds(b, B)], ir, sem)
        pltpu.async_copy(col_hbm.at[pl.ds(b, B)], ic, sem)
        pltpu.async_copy(weight_hbm.at[pl.ds(b, B)], wv, sem)

    def drain_prefetch(s):
        ir, ic, wv, sem = idx[s]
        pltpu.make_async_copy(row_hbm.at[pl.ds(0, B)], ir, sem).wait()
        pltpu.make_async_copy(col_hbm.at[pl.ds(0, B)], ic, sem).wait()
        pltpu.make_async_copy(weight_hbm.at[pl.ds(0, B)], wv, sem).wait()

    def gather(s):
        ir, ic, _, _ = idx[s]
        pack, sem_emb, _ = packs[s]
        pltpu.async_copy(emb_hbm.at[ir], pack.at[:, pl.ds(0, H)], sem_emb)
        pltpu.async_copy(emb_hbm.at[ic], pack.at[:, pl.ds(H, H)], sem_emb)

    def wait_gather(s):
        ir, ic, _, _ = idx[s]
        pack, sem_emb, _ = packs[s]
        pltpu.make_async_copy(
            emb_hbm.at[ir], pack.at[:, pl.ds(0, H)], sem_emb).wait()
        pltpu.make_async_copy(
            emb_hbm.at[ic], pack.at[:, pl.ds(H, H)], sem_emb).wait()

    def drain_out(s):
        pack, _, sem_out = packs[s]
        pltpu.make_async_copy(out_hbm.at[pl.ds(0, B)], pack, sem_out).wait()

    def chunk(g, s, drain):
        pack, _, sem_out = packs[s]
        ir, ic, wv, _ = idx[s]
        base = base0 + g * B

        for grp in range(B // L):
            e0 = grp * L
            ni = ir[pl.ds(e0, L)] * NF6
            nj = ic[pl.ds(e0, L)] * NF6

            def gcol(nidx, c):
                return plsc.load_gather(feat6, [nidx + c])

            fa = [gcol(ni, c) for c in range(6)]
            fb = [gcol(nj, c) for c in range(6)]
            sim = fa[0] * fb[0] + fa[1] * fb[1] + fa[2] * fb[2] + fa[3] * fb[3]
            dx = fa[4] - fb[4]
            dy = fa[5] - fb[5]
            r = _rsqrt(dx * dx + dy * dy + 1e-12)
            w = jnp.abs(wv[pl.ds(e0, L)])
            vals = [w, sim, dx, dy, jnp.abs(dx), jnp.abs(dy), dx * r, dy * r]
            ei = lax.iota(jnp.int32, L) + e0
            for k, v in enumerate(vals):
                kk = jnp.full((L,), 2 * H + k, jnp.int32)
                plsc.store_scatter(pack, [ei, kk], v)

        if drain:
            drain_out(1 - s)
        drain_prefetch(1 - s)
        gather(1 - s)
        wait_gather(s)
        prefetch(g + 2, s)
        pltpu.async_copy(pack, out_hbm.at[pl.ds(base, B)], sem_out)

    prefetch(0, 0)
    drain_prefetch(0)
    gather(0)
    prefetch(1, 1)

    chunk(0, 0, drain=False)

    def pair(p, carry):
        chunk(2 * p + 1, 1, drain=True)
        chunk(2 * p + 2, 0, drain=True)
        return carry

    lax.fori_loop(0, (NCHUNK - 1) // 2, pair, 0)

    drain_out(0)
    wait_gather(1)
    drain_prefetch(0)


@jax.jit
def _encode(node_embeddings, row, col, edge_weight, feat6):
    mesh = plsc.VectorSubcoreMesh(core_axis_name="c", subcore_axis_name="s")
    k = pl.kernel(
        _edge_body,
        out_type=jax.ShapeDtypeStruct((E, OUT_D), jnp.float32),
        mesh=mesh,
        scratch_types=[
            pltpu.VMEM((B,), jnp.int32),
            pltpu.VMEM((B,), jnp.int32),
            pltpu.VMEM((B,), jnp.int32),
            pltpu.VMEM((B,), jnp.int32),
            pltpu.VMEM((B,), jnp.float32),
            pltpu.VMEM((B,), jnp.float32),
            pltpu.VMEM((N * NF6,), jnp.float32),
            pltpu.VMEM((B, OUT_D), jnp.float32),
            pltpu.VMEM((B, OUT_D), jnp.float32),
            pltpu.SemaphoreType.DMA,
            pltpu.SemaphoreType.DMA,
            pltpu.SemaphoreType.DMA,
            pltpu.SemaphoreType.DMA,
            pltpu.SemaphoreType.DMA,
            pltpu.SemaphoreType.DMA,
        ],
        compiler_params=pltpu.CompilerParams(needs_layout_passes=False),
    )
    return k(row, col, edge_weight, node_embeddings, feat6)


def kernel(node_embeddings, edge_index, edge_weight, node_features):
    row = edge_index[0]
    col = edge_index[1]
    feat6 = node_features[:, :NF6].reshape(-1)
    return _encode(node_embeddings, row, col, edge_weight, feat6)


# speedup vs baseline: 1.0227x; 1.0227x over previous
import functools

import jax
import jax.numpy as jnp
from jax import lax
from jax.experimental import pallas as pl
from jax.experimental.pallas import tpu as pltpu
from jax.experimental.pallas import tpu_sc as plsc

N = 10000
E = 320000
H = 128
NF6 = 6
OUT_D = 264

NC = 2
NS = 16
NW = NC * NS
EPW = E // NW
B = 80
NCHUNK = EPW // B
L = 16


def _rsqrt(x):
    xi = lax.bitcast_convert_type(x, jnp.int32)
    yi = jnp.int32(0x5F3759DF) - lax.shift_right_logical(xi, 1)
    y = lax.bitcast_convert_type(yi, jnp.float32)
    xh = x * 0.5
    for _ in range(3):
        y = y * (1.5 - xh * y * y)
    return y


def _edge_body(row_hbm, col_hbm, weight_hbm, emb_hbm, feat6_hbm, out_hbm,
               ir0, ir1, ic0, ic1, wv0, wv1, feat6, pack0, pack1,
               sem_emb0, sem_emb1, sem_out0, sem_out1, sem_i0, sem_i1):
    wid = lax.axis_index("s") * NC + lax.axis_index("c")
    base0 = wid * EPW
    pltpu.sync_copy(feat6_hbm, feat6)

    idx = [(ir0, ic0, wv0, sem_i0), (ir1, ic1, wv1, sem_i1)]
    packs = [(pack0, sem_emb0, sem_out0), (pack1, sem_emb1, sem_out1)]

    def prefetch(g, s):
        ir, ic, wv, sem = idx[s]
        b = jnp.minimum(base0 + g * B, base0 + EPW - B)
        pltpu.async_copy(row_hbm.at[pl.ds(b, B)], ir, sem)
        pltpu.async_copy(col_hbm.at[pl.ds(b, B)], ic, sem)
        pltpu.async_copy(weight_hbm.at[pl.ds(b, B)], wv, sem)

    def drain_prefetch(s):
        ir, ic, wv, sem = idx[s]
        pltpu.make_async_copy(row_hbm.at[pl.ds(0, B)], ir, sem).wait()
        pltpu.make_async_copy(col_hbm.at[pl.ds(0, B)], ic, sem).wait()
        pltpu.make_async_copy(weight_hbm.at[pl.ds(0, B)], wv, sem).wait()

    def gather(s):
        ir, ic, _, _ = idx[s]
        pack, sem_emb, _ = packs[s]
        pltpu.async_copy(emb_hbm.at[ir], pack.at[:, pl.ds(0, H)], sem_emb)
        pltpu.async_copy(emb_hbm.at[ic], pack.at[:, pl.ds(H, H)], sem_emb)

    def wait_gather(s):
        ir, ic, _, _ = idx[s]
        pack, sem_emb, _ = packs[s]
        pltpu.make_async_copy(
            emb_hbm.at[ir], pack.at[:, pl.ds(0, H)], sem_emb).wait()
        pltpu.make_async_copy(
            emb_hbm.at[ic], pack.at[:, pl.ds(H, H)], sem_emb).wait()

    def drain_out(s):
        pack, _, sem_out = packs[s]
        pltpu.make_async_copy(out_hbm.at[pl.ds(0, B)], pack, sem_out).wait()

    def chunk(g, s, drain):
        pack, _, sem_out = packs[s]
        ir, ic, wv, _ = idx[s]
        base = base0 + g * B

        for grp in range(B // L):
            e0 = grp * L
            ni = ir[pl.ds(e0, L)] * NF6
            nj = ic[pl.ds(e0, L)] * NF6

            def gcol(nidx, c):
                return plsc.load_gather(feat6, [nidx + c])

            fa = [gcol(ni, c) for c in range(6)]
            fb = [gcol(nj, c) for c in range(6)]
            dot = fa[0] * fb[0] + fa[1] * fb[1] + fa[2] * fb[2] + fa[3] * fb[3]
            si = fa[0] * fa[0] + fa[1] * fa[1] + fa[2] * fa[2] + fa[3] * fa[3]
            sj = fb[0] * fb[0] + fb[1] * fb[1] + fb[2] * fb[2] + fb[3] * fb[3]
            sim = dot * _rsqrt(jnp.maximum(si, 1e-16)) * _rsqrt(jnp.maximum(sj, 1e-16))
            dx = fa[4] - fb[4]
            dy = fa[5] - fb[5]
            r = _rsqrt(dx * dx + dy * dy + 1e-12)
            w = jnp.abs(wv[pl.ds(e0, L)])
            vals = [w, sim, dx, dy, jnp.abs(dx), jnp.abs(dy), dx * r, dy * r]
            ei = lax.iota(jnp.int32, L) + e0
            for k, v in enumerate(vals):
                kk = jnp.full((L,), 2 * H + k, jnp.int32)
                plsc.store_scatter(pack, [ei, kk], v)

        if drain:
            drain_out(1 - s)
        drain_prefetch(1 - s)
        gather(1 - s)
        wait_gather(s)
        prefetch(g + 2, s)
        pltpu.async_copy(pack, out_hbm.at[pl.ds(base, B)], sem_out)

    prefetch(0, 0)
    drain_prefetch(0)
    gather(0)
    prefetch(1, 1)

    chunk(0, 0, drain=False)

    def pair(p, carry):
        chunk(2 * p + 1, 1, drain=True)
        chunk(2 * p + 2, 0, drain=True)
        return carry

    lax.fori_loop(0, (NCHUNK - 1) // 2, pair, 0)

    drain_out(0)
    wait_gather(1)
    drain_prefetch(0)


@jax.jit
def _encode(node_embeddings, row, col, edge_weight, feat6):
    mesh = plsc.VectorSubcoreMesh(core_axis_name="c", subcore_axis_name="s")
    k = pl.kernel(
        _edge_body,
        out_type=jax.ShapeDtypeStruct((E, OUT_D), jnp.float32),
        mesh=mesh,
        scratch_types=[
            pltpu.VMEM((B,), jnp.int32),
            pltpu.VMEM((B,), jnp.int32),
            pltpu.VMEM((B,), jnp.int32),
            pltpu.VMEM((B,), jnp.int32),
            pltpu.VMEM((B,), jnp.float32),
            pltpu.VMEM((B,), jnp.float32),
            pltpu.VMEM((N * NF6,), jnp.float32),
            pltpu.VMEM((B, OUT_D), jnp.float32),
            pltpu.VMEM((B, OUT_D), jnp.float32),
            pltpu.SemaphoreType.DMA,
            pltpu.SemaphoreType.DMA,
            pltpu.SemaphoreType.DMA,
            pltpu.SemaphoreType.DMA,
            pltpu.SemaphoreType.DMA,
            pltpu.SemaphoreType.DMA,
        ],
        compiler_params=pltpu.CompilerParams(needs_layout_passes=False),
    )
    return k(row, col, edge_weight, node_embeddings, feat6)


def kernel(node_embeddings, edge_index, edge_weight, node_features):
    row = edge_index[0]
    col = edge_index[1]
    feat6 = node_features[:, :NF6].reshape(-1)
    return _encode(node_embeddings, row, col, edge_weight, feat6)
